# baseline (speedup 1.0000x reference)
import functools

import jax
import jax.numpy as jnp
from jax import lax
from jax.experimental import pallas as pl
from jax.experimental.pallas import tpu as pltpu
from jax.experimental.pallas import tpu_sc as plsc

LANES = 16
IDX_CHUNK = 128


def _sc_dims():
    try:
        info = plsc.get_sparse_core_info()
        return info.num_cores, info.num_subcores
    except Exception:
        return 2, 16


def _make_body(nc, nchunk, bpw):
    def body(users_hbm, items_hbm, uf_hbm, if_hbm, out_hbm,
             idx_u, idx_v, u_rows, v_rows, out_v, sem):
        wid = lax.axis_index("s") * nc + lax.axis_index("c")
        base = wid * bpw

        cp_u = pltpu.async_copy(users_hbm.at[wid], idx_u, sem)
        cp_v = pltpu.async_copy(items_hbm.at[wid], idx_v, sem)
        cp_u.wait()
        cp_v.wait()

        handles = []
        for j in range(nchunk):
            dst = u_rows.at[pl.ds(j * IDX_CHUNK, IDX_CHUNK)]
            handles.append(pltpu.async_copy(uf_hbm.at[idx_u.at[j]], dst, sem))
            dst = v_rows.at[pl.ds(j * IDX_CHUNK, IDX_CHUNK)]
            handles.append(pltpu.async_copy(if_hbm.at[idx_v.at[j]], dst, sem))
        for h in handles:
            h.wait()

        lane = lax.iota(jnp.int32, LANES)
        perms = [lane ^ d for d in (1, 2, 4, 8)]

        def blk(b, _):
            acc = jnp.zeros((LANES,), jnp.float32)
            for j in range(LANES):
                r = b * LANES + j
                p = u_rows[r] * v_rows[r]
                for perm in perms:
                    p = p + p.at[perm].get(mode="promise_in_bounds")
                acc = jnp.where(lane == j, p, acc)
            out_v[pl.ds(b * LANES, LANES)] = acc
            return 0

        lax.fori_loop(0, bpw // LANES, blk, 0)
        pltpu.sync_copy(out_v, out_hbm.at[pl.ds(base, bpw)])

    return body


@jax.jit
def kernel(x, user_factors, item_factors):
    nc, ns = _sc_dims()
    nw = nc * ns
    batch = x.shape[0]
    assert batch % (nw * IDX_CHUNK) == 0
    bpw = batch // nw
    nchunk = bpw // IDX_CHUNK

    users = x[:, 0].astype(jnp.int32).reshape(nw, nchunk, IDX_CHUNK)
    items = x[:, 1].astype(jnp.int32).reshape(nw, nchunk, IDX_CHUNK)

    mesh = plsc.VectorSubcoreMesh(core_axis_name="c", subcore_axis_name="s")
    fn = pl.kernel(
        _make_body(nc, nchunk, bpw),
        out_type=jax.ShapeDtypeStruct((batch,), jnp.float32),
        mesh=mesh,
        scratch_types=[
            pltpu.VMEM((nchunk, IDX_CHUNK), jnp.int32),
            pltpu.VMEM((nchunk, IDX_CHUNK), jnp.int32),
            pltpu.VMEM((bpw, LANES), jnp.float32),
            pltpu.VMEM((bpw, LANES), jnp.float32),
            pltpu.VMEM((bpw,), jnp.float32),
            pltpu.SemaphoreType.DMA,
        ],
        compiler_params=pltpu.CompilerParams(use_tc_tiling_on_sc=False),
    )
    return fn(users, items, user_factors, item_factors)
